# baseline (speedup 1.0000x reference)
import functools

import jax
import jax.numpy as jnp
from jax import lax
from jax.experimental import pallas as pl
from jax.experimental.pallas import tpu as pltpu
from jax.experimental.pallas import tpu_sc as plsc

B = 4096
L = 50
D = 64
TOK = 128


@functools.cache
def _build():
    info = plsc.get_sparse_core_info()
    NC, NS = info.num_cores, info.num_subcores
    NW = NC * NS
    n_per_w = B * L // NW
    g_per_w = n_per_w // TOK
    mesh = plsc.VectorSubcoreMesh(core_axis_name="c", subcore_axis_name="s")

    @functools.partial(
        pl.kernel,
        mesh=mesh,
        out_type=jax.ShapeDtypeStruct((L, D, B), jnp.float32),
        scratch_types=[
            pltpu.VMEM((g_per_w, TOK), jnp.int32),
            pltpu.VMEM((TOK, D), jnp.float32),
            pltpu.VMEM((TOK, D), jnp.float32),
            pltpu.VMEM((D, TOK), jnp.float32),
            pltpu.VMEM((D, TOK), jnp.float32),
        ]
        + [pltpu.SemaphoreType.DMA] * 4,
        compiler_params=pltpu.CompilerParams(
            use_tc_tiling_on_sc=True, needs_layout_passes=False),
    )
    def emb(idx_hbm, tab_hbm, out_hbm, idx_v, rows0, rows1, blk0, blk1,
            gsem0, gsem1, ssem0, ssem1):
        wid = lax.axis_index("s") * NC + lax.axis_index("c")
        t0w = wid * n_per_w
        pltpu.sync_copy(idx_hbm.at[wid], idx_v)
        lanes = lax.iota(jnp.int32, 16)
        lanesjg = [lanes + 16 * jg for jg in range(8)]

        def issue_fetch(g, rows, gsem):
            for jg in range(8):
                v = idx_v[g, pl.ds(jg * 16, 16)]
                for k in range(16):
                    pltpu.async_copy(
                        tab_hbm.at[pl.ds(v[k], 1)],
                        rows.at[pl.ds(jg * 16 + k, 1)], gsem)

        def drain_fetch(rows, gsem):
            pltpu.make_async_copy(
                tab_hbm.at[pl.ds(0, TOK)], rows, gsem).wait()

        def transpose(rows, blk):
            def tbody(dg, c):
                for du in range(4):
                    d = dg * 4 + du
                    dv = lax.broadcast(d, (16,))
                    for jg in range(8):
                        blk[d, pl.ds(jg * 16, 16)] = plsc.load_gather(
                            rows, [lanesjg[jg], dv])
                return c
            lax.fori_loop(0, D // 4, tbody, 0)

        def store_slice(g, blk, ssem, wait):
            t0 = t0w + g * TOK
            l = t0 >> 12
            b0 = pl.multiple_of(t0 & (B - 1), TOK)
            dst = out_hbm.at[l, pl.ds(0, D), pl.ds(b0, TOK)]
            if wait:
                pltpu.make_async_copy(blk, dst, ssem).wait()
            else:
                pltpu.async_copy(blk, dst, ssem)

        issue_fetch(0, rows0, gsem0)

        def pair(i, c):
            g0 = 2 * i
            drain_fetch(rows0, gsem0)
            issue_fetch(g0 + 1, rows1, gsem1)

            @pl.when(i > 0)
            def _():
                store_slice(g0 - 2, blk0, ssem0, wait=True)

            transpose(rows0, blk0)
            store_slice(g0, blk0, ssem0, wait=False)

            drain_fetch(rows1, gsem1)

            @pl.when(g0 + 2 < g_per_w)
            def _():
                issue_fetch(g0 + 2, rows0, gsem0)

            @pl.when(i > 0)
            def _():
                store_slice(g0 - 1, blk1, ssem1, wait=True)

            transpose(rows1, blk1)
            store_slice(g0 + 1, blk1, ssem1, wait=False)
            return c

        lax.fori_loop(0, g_per_w // 2, pair, 0)
        store_slice(g_per_w - 2, blk0, ssem0, wait=True)
        store_slice(g_per_w - 1, blk1, ssem1, wait=True)

    return emb


def kernel(x, table):
    info = plsc.get_sparse_core_info()
    NW = info.num_cores * info.num_subcores
    idx = x.astype(jnp.int32).T.reshape(NW, B * L // NW // TOK, TOK)
    out = _build()(idx, table)
    return out.transpose(2, 0, 1)


# speedup vs baseline: 1.0026x; 1.0026x over previous
import functools

import jax
import jax.numpy as jnp
from jax import lax
from jax.experimental import pallas as pl
from jax.experimental.pallas import tpu as pltpu
from jax.experimental.pallas import tpu_sc as plsc

B = 4096
L = 50
D = 64
TOK = 128
NR = 4


@functools.cache
def _build():
    info = plsc.get_sparse_core_info()
    NC, NS = info.num_cores, info.num_subcores
    NW = NC * NS
    n_per_w = B * L // NW
    g_per_w = n_per_w // TOK
    n_quads = (g_per_w - 2) // NR
    mesh = plsc.VectorSubcoreMesh(core_axis_name="c", subcore_axis_name="s")

    @functools.partial(
        pl.kernel,
        mesh=mesh,
        out_type=jax.ShapeDtypeStruct((L, D, B), jnp.float32),
        scratch_types=[
            pltpu.VMEM((g_per_w, TOK), jnp.int32),
        ]
        + [pltpu.VMEM((TOK, D), jnp.float32)] * NR
        + [pltpu.VMEM((D, TOK), jnp.float32)] * 2
        + [pltpu.SemaphoreType.DMA] * (NR + 2),
        compiler_params=pltpu.CompilerParams(
            use_tc_tiling_on_sc=True, needs_layout_passes=False),
    )
    def emb(idx_hbm, tab_hbm, out_hbm, idx_v, *bufs):
        rows = bufs[:NR]
        blks = bufs[NR:NR + 2]
        gsems = bufs[NR + 2:2 * NR + 2]
        ssems = bufs[2 * NR + 2:]
        wid = lax.axis_index("s") * NC + lax.axis_index("c")
        t0w = wid * n_per_w
        pltpu.sync_copy(idx_hbm.at[wid], idx_v)
        lanes = lax.iota(jnp.int32, 16)
        lanesjg = [lanes + 16 * jg for jg in range(8)]

        def issue_fetch(g, s):
            def fbody(jg, c):
                v = idx_v[g, pl.ds(jg * 16, 16)]
                for k in range(16):
                    pltpu.async_copy(
                        tab_hbm.at[pl.ds(v[k], 1)],
                        rows[s].at[pl.ds(jg * 16 + k, 1)], gsems[s])
                return c
            lax.fori_loop(0, 8, fbody, 0)

        def drain_fetch(s):
            pltpu.make_async_copy(
                tab_hbm.at[pl.ds(0, TOK)], rows[s], gsems[s]).wait()

        def transpose(s, p):
            def tbody(dg, c):
                for du in range(4):
                    d = dg * 4 + du
                    dv = lax.broadcast(d, (16,))
                    for jg in range(8):
                        blks[p][d, pl.ds(jg * 16, 16)] = plsc.load_gather(
                            rows[s], [lanesjg[jg], dv])
                return c
            lax.fori_loop(0, D // 4, tbody, 0)

        def store_slice(g, p, wait):
            t0 = t0w + g * TOK
            l = t0 >> 12
            b0 = pl.multiple_of(t0 & (B - 1), TOK)
            dst = out_hbm.at[l, pl.ds(0, D), pl.ds(b0, TOK)]
            if wait:
                pltpu.make_async_copy(blks[p], dst, ssems[p]).wait()
            else:
                pltpu.async_copy(blks[p], dst, ssems[p])

        for s in range(NR):
            issue_fetch(s, s)

        def quad(i, c):
            g0 = NR * i
            for s in range(NR):
                g = g0 + s
                p = s % 2
                drain_fetch(s)

                @pl.when(g >= 2)
                def _():
                    store_slice(g - 2, p, wait=True)

                transpose(s, p)

                @pl.when(g + NR < g_per_w)
                def _():
                    issue_fetch(g + NR, s)

                store_slice(g, p, wait=False)
            return c

        lax.fori_loop(0, n_quads, quad, 0)
        for s, g in ((0, g_per_w - 2), (1, g_per_w - 1)):
            p = s % 2
            drain_fetch(s)
            store_slice(g - 2, p, wait=True)
            transpose(s, p)
            store_slice(g, p, wait=False)
        store_slice(g_per_w - 2, 0, wait=True)
        store_slice(g_per_w - 1, 1, wait=True)

    return emb


def kernel(x, table):
    info = plsc.get_sparse_core_info()
    NW = info.num_cores * info.num_subcores
    idx = x.astype(jnp.int32).T.reshape(NW, B * L // NW // TOK, TOK)
    out = _build()(idx, table)
    return out.transpose(2, 0, 1)


# speedup vs baseline: 1.4661x; 1.4623x over previous
import functools

import jax
import jax.numpy as jnp
from jax import lax
from jax.experimental import pallas as pl
from jax.experimental.pallas import tpu as pltpu
from jax.experimental.pallas import tpu_sc as plsc

B = 4096
L = 50
D = 64
TOK = 128
NR = 5


@functools.cache
def _build():
    info = plsc.get_sparse_core_info()
    NC, NS = info.num_cores, info.num_subcores
    NW = NC * NS
    n_per_w = B * L // NW
    g_per_w = n_per_w // TOK
    mesh = plsc.VectorSubcoreMesh(core_axis_name="c", subcore_axis_name="s")

    @functools.partial(
        pl.kernel,
        mesh=mesh,
        out_type=jax.ShapeDtypeStruct((L, B, D), jnp.float32),
        scratch_types=[
            pltpu.VMEM((g_per_w, TOK), jnp.int32),
        ]
        + [pltpu.VMEM((TOK, D), jnp.float32)] * NR
        + [pltpu.SemaphoreType.DMA] * (2 * NR),
        compiler_params=pltpu.CompilerParams(use_tc_tiling_on_sc=True),
    )
    def emb(idx_hbm, tab_hbm, out_hbm, idx_v, *bufs):
        rows = bufs[:NR]
        gsems = bufs[NR:2 * NR]
        ssems = bufs[2 * NR:]
        wid = lax.axis_index("s") * NC + lax.axis_index("c")
        t0w = wid * n_per_w
        pltpu.sync_copy(idx_hbm.at[wid], idx_v)

        def issue_fetch(g, s):
            def fbody(jg, c):
                v = idx_v[g, pl.ds(jg * 16, 16)]
                for k in range(16):
                    pltpu.async_copy(
                        tab_hbm.at[pl.ds(v[k], 1)],
                        rows[s].at[pl.ds(jg * 16 + k, 1)], gsems[s])
                return c
            lax.fori_loop(0, 8, fbody, 0)

        def drain_fetch(s):
            pltpu.make_async_copy(
                tab_hbm.at[pl.ds(0, TOK)], rows[s], gsems[s]).wait()

        def out_dst(g):
            t0 = t0w + g * TOK
            l = t0 >> 12
            b0 = pl.multiple_of(t0 & (B - 1), TOK)
            return out_hbm.at[l, pl.ds(b0, TOK), pl.ds(0, D)]

        def store_block(g, s):
            pltpu.async_copy(rows[s], out_dst(g), ssems[s])

        def drain_store(g, s):
            pltpu.make_async_copy(rows[s], out_dst(g), ssems[s]).wait()

        for s in range(NR):
            issue_fetch(s, s)

        def ring(i, c):
            g0 = NR * i
            for s in range(NR):
                g = g0 + s
                sprev = (s - 1) % NR
                drain_fetch(s)
                store_block(g, s)

                @pl.when(g >= 1)
                def _():
                    drain_store(g - 1, sprev)

                @pl.when((g >= 1) & (g - 1 + NR < g_per_w))
                def _():
                    issue_fetch(g - 1 + NR, sprev)

            return c

        lax.fori_loop(0, g_per_w // NR, ring, 0)
        drain_store(g_per_w - 1, NR - 1)

    return emb


def kernel(x, table):
    info = plsc.get_sparse_core_info()
    NW = info.num_cores * info.num_subcores
    idx = x.astype(jnp.int32).T.reshape(NW, B * L // NW // TOK, TOK)
    out = _build()(idx, table)
    return out.transpose(1, 0, 2)
